# Initial kernel scaffold; baseline (speedup 1.0000x reference)
import jax
import jax.numpy as jnp
from jax.experimental import pallas as pl


def kernel(x, edge_index, edge_feature, W1, b1, W2, b2, W3, b3):
    raise NotImplementedError("write your pallas kernel here")


# speedup vs baseline: 7.8647x; 7.8647x over previous
import functools

import jax
import jax.numpy as jnp
from jax import lax
from jax.experimental import pallas as pl
from jax.experimental.pallas import tpu as pltpu
from jax.experimental.pallas import tpu_sc as plsc

N = 10000
E = 320000
NC, NS = 2, 16
NW = NC * NS
EPW = E // NW
KB = 80
NBATCH = EPW // KB
ZR = 624

_MESH = plsc.VectorSubcoreMesh(core_axis_name="c", subcore_axis_name="s")
_SC_PARAMS = pltpu.CompilerParams(needs_layout_passes=False)
_HIGH = jax.lax.Precision.HIGHEST


def _sc_deg_body(dst_hbm, ew_hbm, zn_hbm, out_hbm, dbuf, wbuf, stage, acc_sp):
    c = lax.axis_index("c")
    s = lax.axis_index("s")
    wid = s * NC + c

    @pl.when(s == 0)
    def _():
        pltpu.sync_copy(zn_hbm, stage)
        pltpu.sync_copy(stage, acc_sp)

    plsc.subcore_barrier()
    base = wid * EPW

    def body(i, carry):
        off = base + i * KB
        pltpu.sync_copy(dst_hbm.at[pl.ds(off, KB)], dbuf)
        pltpu.sync_copy(ew_hbm.at[pl.ds(off, KB)], wbuf)
        pltpu.sync_copy(wbuf, acc_sp.at[dbuf], add=True)
        return carry

    lax.fori_loop(0, NBATCH, body, 0)
    plsc.subcore_barrier()

    @pl.when(s == 0)
    def _():
        pltpu.sync_copy(acc_sp, stage)
        pltpu.sync_copy(stage, out_hbm.at[pl.ds(pl.multiple_of(c * N, 8), N)])


def _sc_norm_body(src_hbm, dst_hbm, ew_hbm, dinv_hbm, out_hbm,
                  dv_v, sbuf, dbuf, wbuf, nbuf):
    c = lax.axis_index("c")
    s = lax.axis_index("s")
    wid = s * NC + c
    pltpu.sync_copy(dinv_hbm, dv_v)
    base = wid * EPW

    def body(i, carry):
        off = base + i * KB
        pltpu.sync_copy(src_hbm.at[pl.ds(off, KB)], sbuf)
        pltpu.sync_copy(dst_hbm.at[pl.ds(off, KB)], dbuf)
        pltpu.sync_copy(ew_hbm.at[pl.ds(off, KB)], wbuf)
        for j in range(KB // 16):
            sl = pl.ds(j * 16, 16)
            a = plsc.load_gather(dv_v, [sbuf[sl]])
            b = plsc.load_gather(dv_v, [dbuf[sl]])
            nbuf[sl] = a * wbuf[sl] * b
        pltpu.sync_copy(nbuf, out_hbm.at[pl.ds(off, KB)])
        return carry

    lax.fori_loop(0, NBATCH, body, 0)


def _sc_agg_body(y_hbm, src_hbm, dst_hbm, nrm_hbm, z_hbm, out_hbm,
                 sbuf, dbuf, nbuf, rows, acc_sp, sem):
    c = lax.axis_index("c")
    s = lax.axis_index("s")
    wid = s * NC + c
    start = pl.multiple_of(s * ZR, 8)

    @pl.when(s < NS - 1)
    def _():
        pltpu.sync_copy(z_hbm.at[pl.ds(0, ZR)], acc_sp.at[pl.ds(start, ZR)])

    @pl.when(s == NS - 1)
    def _():
        pltpu.sync_copy(z_hbm, acc_sp.at[pl.ds(start, N - (NS - 1) * ZR)])

    plsc.subcore_barrier()
    base = wid * EPW

    def body(i, carry):
        off = base + i * KB
        pltpu.sync_copy(src_hbm.at[pl.ds(off, KB)], sbuf)
        pltpu.sync_copy(dst_hbm.at[pl.ds(off, KB)], dbuf)
        pltpu.sync_copy(nrm_hbm.at[pl.ds(off, KB)], nbuf)
        pltpu.async_copy(y_hbm.at[sbuf], rows, sem).wait()

        def scale(r, cr):
            nb = plsc.load_gather(nbuf, [jnp.full((16,), r, jnp.int32)])
            for cc in range(8):
                sl = pl.ds(cc * 16, 16)
                rows[r, sl] = rows[r, sl] * nb
            return cr

        lax.fori_loop(0, KB, scale, 0)
        pltpu.sync_copy(rows, acc_sp.at[dbuf], add=True)
        return carry

    lax.fori_loop(0, NBATCH, body, 0)
    plsc.subcore_barrier()
    ostart = pl.multiple_of(c * N + s * ZR, 8)

    @pl.when(s < NS - 1)
    def _():
        pltpu.sync_copy(acc_sp.at[pl.ds(start, ZR)], out_hbm.at[pl.ds(ostart, ZR)])

    @pl.when(s == NS - 1)
    def _():
        last = N - (NS - 1) * ZR
        pltpu.sync_copy(acc_sp.at[pl.ds(start, last)],
                        out_hbm.at[pl.ds(ostart, last)])


def _sc_agg1_body(y_hbm, src_hbm, dst_hbm, nrm_hbm, zn_hbm, out_hbm,
                  y_v, sbuf, dbuf, nbuf, vbuf, stage, acc_sp):
    c = lax.axis_index("c")
    s = lax.axis_index("s")
    wid = s * NC + c
    pltpu.sync_copy(y_hbm, y_v)

    @pl.when(s == 0)
    def _():
        pltpu.sync_copy(zn_hbm, stage)
        pltpu.sync_copy(stage, acc_sp)

    plsc.subcore_barrier()
    base = wid * EPW

    def body(i, carry):
        off = base + i * KB
        pltpu.sync_copy(src_hbm.at[pl.ds(off, KB)], sbuf)
        pltpu.sync_copy(dst_hbm.at[pl.ds(off, KB)], dbuf)
        pltpu.sync_copy(nrm_hbm.at[pl.ds(off, KB)], nbuf)
        for j in range(KB // 16):
            sl = pl.ds(j * 16, 16)
            vbuf[sl] = plsc.load_gather(y_v, [sbuf[sl]]) * nbuf[sl]
        pltpu.sync_copy(vbuf, acc_sp.at[dbuf], add=True)
        return carry

    lax.fori_loop(0, NBATCH, body, 0)
    plsc.subcore_barrier()

    @pl.when(s == 0)
    def _():
        pltpu.sync_copy(acc_sp, stage)
        pltpu.sync_copy(stage, out_hbm.at[pl.ds(pl.multiple_of(c * N, 8), N)])


_sc_deg = functools.partial(
    pl.kernel, _sc_deg_body, mesh=_MESH,
    compiler_params=_SC_PARAMS,
    out_type=jax.ShapeDtypeStruct((NC * N,), jnp.float32),
    scratch_types=[
        pltpu.VMEM((KB,), jnp.int32),
        pltpu.VMEM((KB,), jnp.float32),
        pltpu.VMEM((N,), jnp.float32),
        pltpu.VMEM_SHARED((N,), jnp.float32),
    ],
)

_sc_norm = functools.partial(
    pl.kernel, _sc_norm_body, mesh=_MESH,
    compiler_params=_SC_PARAMS,
    out_type=jax.ShapeDtypeStruct((E,), jnp.float32),
    scratch_types=[
        pltpu.VMEM((N,), jnp.float32),
        pltpu.VMEM((KB,), jnp.int32),
        pltpu.VMEM((KB,), jnp.int32),
        pltpu.VMEM((KB,), jnp.float32),
        pltpu.VMEM((KB,), jnp.float32),
    ],
)

_sc_agg = functools.partial(
    pl.kernel, _sc_agg_body, mesh=_MESH,
    compiler_params=_SC_PARAMS,
    out_type=jax.ShapeDtypeStruct((NC * N, 128), jnp.float32),
    scratch_types=[
        pltpu.VMEM((KB,), jnp.int32),
        pltpu.VMEM((KB,), jnp.int32),
        pltpu.VMEM((KB,), jnp.float32),
        pltpu.VMEM((KB, 128), jnp.float32),
        pltpu.VMEM_SHARED((N, 128), jnp.float32),
        pltpu.SemaphoreType.DMA,
    ],
)

_sc_agg1 = functools.partial(
    pl.kernel, _sc_agg1_body, mesh=_MESH,
    compiler_params=_SC_PARAMS,
    out_type=jax.ShapeDtypeStruct((NC * N,), jnp.float32),
    scratch_types=[
        pltpu.VMEM((N,), jnp.float32),
        pltpu.VMEM((KB,), jnp.int32),
        pltpu.VMEM((KB,), jnp.int32),
        pltpu.VMEM((KB,), jnp.float32),
        pltpu.VMEM((KB,), jnp.float32),
        pltpu.VMEM((N,), jnp.float32),
        pltpu.VMEM_SHARED((N,), jnp.float32),
    ],
)


def _ew_body(ef_ref, m_ref, o_ref):
    o_ref[...] = jnp.dot(ef_ref[...], m_ref[...],
                         preferred_element_type=jnp.float32, precision=_HIGH)


def _dinv_body(degp_ref, o_ref):
    deg = degp_ref[0:1, :] + degp_ref[1:2, :] + 1.0
    o_ref[...] = jnp.where(deg > 0, lax.rsqrt(deg), 0.0)


def _l1_body(p0, p1, xr, dv, w1, b1r, w2, o):
    d2 = dv[...] * dv[...]
    z = p0[...] + p1[...] + d2 * xr[...]
    h = jnp.dot(z, w1[...], preferred_element_type=jnp.float32, precision=_HIGH)
    h = jnp.maximum(h + b1r[...], 0.0)
    o[...] = jnp.dot(h, w2[...], preferred_element_type=jnp.float32,
                     precision=_HIGH)


def _l2_body(p0, p1, xr, dv, b2r, w3, o):
    d2 = dv[...] * dv[...]
    h = jnp.maximum(p0[...] + p1[...] + d2 * xr[...] + b2r[...], 0.0)
    o[...] = jnp.dot(h, w3[...], preferred_element_type=jnp.float32,
                     precision=_HIGH)


def _out_body(p3, xw3r, dv, b3r, o):
    o[...] = (p3[0:1, :] + p3[1:2, :]
              + dv[...] * dv[...] * xw3r[...] + b3r[...])


def kernel(x, edge_index, edge_feature, W1, b1, W2, b2, W3, b3):
    src = edge_index[0]
    dst = edge_index[1]
    zn = jnp.zeros((N,), jnp.float32)
    zrows = jnp.zeros((N - (NS - 1) * ZR, 128), jnp.float32)

    ef2 = edge_feature.reshape(E // 8, 128)
    m = jnp.repeat(jnp.eye(8, dtype=jnp.float32), 16, axis=0) * (1.0 / 16.0)
    ew8 = pl.pallas_call(
        _ew_body,
        grid=(10,),
        in_specs=[pl.BlockSpec((E // 80, 128), lambda i: (i, 0)),
                  pl.BlockSpec((128, 8), lambda i: (0, 0))],
        out_specs=pl.BlockSpec((E // 80, 8), lambda i: (i, 0)),
        out_shape=jax.ShapeDtypeStruct((E // 8, 8), jnp.float32),
    )(ef2, m)
    ew = ew8.reshape(E)

    degp = _sc_deg()(dst, ew, zn).reshape(NC, N)
    dinv2d = pl.pallas_call(
        _dinv_body,
        out_shape=jax.ShapeDtypeStruct((1, N), jnp.float32),
    )(degp)
    dinv = dinv2d.reshape(N)
    dvcol = dinv2d.reshape(N, 1)

    nrm = _sc_norm()(src, dst, ew, dinv)

    p1_ = _sc_agg()(x, src, dst, nrm, zrows)
    b1r = b1.reshape(1, 256)
    b2r = b2.reshape(1, 128)
    xw2 = pl.pallas_call(
        _l1_body,
        grid=(10,),
        in_specs=[pl.BlockSpec((1000, 128), lambda i: (i, 0)),
                  pl.BlockSpec((1000, 128), lambda i: (i, 0)),
                  pl.BlockSpec((1000, 128), lambda i: (i, 0)),
                  pl.BlockSpec((1000, 1), lambda i: (i, 0)),
                  pl.BlockSpec((128, 256), lambda i: (0, 0)),
                  pl.BlockSpec((1, 256), lambda i: (0, 0)),
                  pl.BlockSpec((256, 128), lambda i: (0, 0))],
        out_specs=pl.BlockSpec((1000, 128), lambda i: (i, 0)),
        out_shape=jax.ShapeDtypeStruct((N, 128), jnp.float32),
    )(p1_[:N], p1_[N:], x, dvcol, W1, b1r, W2)

    p2_ = _sc_agg()(xw2, src, dst, nrm, zrows)
    xw3 = pl.pallas_call(
        _l2_body,
        grid=(10,),
        in_specs=[pl.BlockSpec((1000, 128), lambda i: (i, 0)),
                  pl.BlockSpec((1000, 128), lambda i: (i, 0)),
                  pl.BlockSpec((1000, 128), lambda i: (i, 0)),
                  pl.BlockSpec((1000, 1), lambda i: (i, 0)),
                  pl.BlockSpec((1, 128), lambda i: (0, 0)),
                  pl.BlockSpec((128, 1), lambda i: (0, 0))],
        out_specs=pl.BlockSpec((1000, 1), lambda i: (i, 0)),
        out_shape=jax.ShapeDtypeStruct((N, 1), jnp.float32),
    )(p2_[:N], p2_[N:], xw2, dvcol, b2r, W3)

    p3 = _sc_agg1()(xw3.reshape(N), src, dst, nrm, zn).reshape(NC, N)
    b3r = b3.reshape(1, 1)
    out2d = pl.pallas_call(
        _out_body,
        out_shape=jax.ShapeDtypeStruct((1, N), jnp.float32),
    )(p3, xw3.reshape(1, N), dinv2d, b3r)
    return out2d.reshape(N)


# speedup vs baseline: 21.9111x; 2.7860x over previous
import functools

import jax
import jax.numpy as jnp
from jax import lax
from jax.experimental import pallas as pl
from jax.experimental.pallas import tpu as pltpu
from jax.experimental.pallas import tpu_sc as plsc

N = 10000
E = 320000
NC, NS = 2, 16
NW = NC * NS
EPW = E // NW
KB = 80
NBATCH = EPW // KB
ZR = 624

_MESH = plsc.VectorSubcoreMesh(core_axis_name="c", subcore_axis_name="s")
_SC_PARAMS = pltpu.CompilerParams(needs_layout_passes=False)
_HIGH = jax.lax.Precision.HIGHEST


def _sc_deg_body(dst3_hbm, ew_hbm, zn_hbm, out_hbm, dbuf2, wbuf, stage,
                 acc_sp, ss):
    c = lax.axis_index("c")
    s = lax.axis_index("s")
    wid = s * NC + c

    @pl.when(s == 0)
    def _():
        pltpu.sync_copy(zn_hbm, stage)
        pltpu.sync_copy(stage, acc_sp)

    pltpu.sync_copy(dst3_hbm.at[wid], dbuf2)
    pltpu.sync_copy(ew_hbm.at[pl.ds(pl.multiple_of(wid * EPW, 8), EPW)], wbuf)
    plsc.subcore_barrier()

    def chunk(ch, carry):
        for j in range(5):
            i = ch * 5 + j
            off = pl.multiple_of(i * KB, 8)
            pltpu.async_copy(wbuf.at[pl.ds(off, KB)], acc_sp.at[dbuf2.at[i]],
                             ss, add=True)
        for j in range(5):
            i = ch * 5 + j
            off = pl.multiple_of(i * KB, 8)
            pltpu.make_async_copy(wbuf.at[pl.ds(off, KB)],
                                  acc_sp.at[dbuf2.at[i]], ss).wait()
        return carry

    lax.fori_loop(0, NBATCH // 5, chunk, 0)
    plsc.subcore_barrier()

    @pl.when(s == 0)
    def _():
        pltpu.sync_copy(acc_sp, stage)
        pltpu.sync_copy(stage, out_hbm.at[pl.ds(pl.multiple_of(c * N, 8), N)])


def _sc_norm_body(src_hbm, dst_hbm, ew_hbm, dinv_hbm, out_hbm,
                  dv, sbuf, dbuf, wbuf, nbuf):
    c = lax.axis_index("c")
    s = lax.axis_index("s")
    wid = s * NC + c
    base = pl.multiple_of(wid * EPW, 8)
    pltpu.sync_copy(dinv_hbm, dv)
    pltpu.sync_copy(src_hbm.at[pl.ds(base, EPW)], sbuf)
    pltpu.sync_copy(dst_hbm.at[pl.ds(base, EPW)], dbuf)
    pltpu.sync_copy(ew_hbm.at[pl.ds(base, EPW)], wbuf)

    @plsc.parallel_loop(0, EPW // 16, unroll=4)
    def _(m):
        sl = pl.ds(pl.multiple_of(m * 16, 16), 16)
        a = plsc.load_gather(dv, [sbuf[sl]])
        b = plsc.load_gather(dv, [dbuf[sl]])
        nbuf[sl] = a * wbuf[sl] * b

    pltpu.sync_copy(nbuf, out_hbm.at[pl.ds(base, EPW)])


def _sc_agg_body(y_hbm, src_hbm, dst3_hbm, nrm_hbm, z_hbm, out_hbm,
                 nbuf, dbuf2, sb0, sb1, rows0, rows1, acc_sp,
                 gs0, gs1, ss0, ss1, xs0, xs1):
    c = lax.axis_index("c")
    s = lax.axis_index("s")
    wid = s * NC + c
    base = pl.multiple_of(wid * EPW, 8)
    pltpu.sync_copy(nrm_hbm.at[pl.ds(base, EPW)], nbuf)
    pltpu.sync_copy(dst3_hbm.at[wid], dbuf2)
    start = pl.multiple_of(s * ZR, 8)

    @pl.when(s < NS - 1)
    def _():
        pltpu.sync_copy(z_hbm.at[pl.ds(0, ZR)], acc_sp.at[pl.ds(start, ZR)])

    @pl.when(s == NS - 1)
    def _():
        pltpu.sync_copy(z_hbm, acc_sp.at[pl.ds(start, N - (NS - 1) * ZR)])

    plsc.subcore_barrier()

    def sidx(i):
        return src_hbm.at[pl.ds(pl.multiple_of(base + i * KB, 8), KB)]

    def scale(rows_b, i):
        @plsc.parallel_loop(0, KB, unroll=2)
        def _(r):
            nb = plsc.load_gather(
                nbuf, [jnp.full((16,), i * KB + r, jnp.int32)])
            for cc in range(8):
                sl = pl.ds(cc * 16, 16)
                rows_b[r, sl] = rows_b[r, sl] * nb

    pltpu.sync_copy(sidx(0), sb0)
    pltpu.async_copy(y_hbm.at[sb0], rows0, gs0).wait()
    scale(rows0, 0)
    pltpu.sync_copy(rows0, acc_sp.at[dbuf2.at[0]], add=True)

    pltpu.sync_copy(sidx(1), sb0)
    pltpu.sync_copy(sidx(2), sb1)
    pltpu.async_copy(y_hbm.at[sb0], rows0, gs0)
    pltpu.async_copy(y_hbm.at[sb1], rows1, gs1)

    def stage_b(i, rows_b, sb_b, gs_b, ss_b, xs_b):
        pltpu.make_async_copy(y_hbm.at[sb_b], rows_b, gs_b).wait()

        @pl.when(i + 2 < NBATCH)
        def _():
            pltpu.async_copy(sidx(i + 2), sb_b, xs_b)

        scale(rows_b, i)
        pltpu.async_copy(rows_b, acc_sp.at[dbuf2.at[i]], ss_b, add=True)

    def drain_b(i, rows_b, sb_b, gs_b, ss_b, xs_b):
        pltpu.make_async_copy(rows_b, acc_sp.at[dbuf2.at[i]], ss_b).wait()

        @pl.when(i + 2 < NBATCH)
        def _():
            pltpu.make_async_copy(sidx(i + 2), sb_b, xs_b).wait()
            pltpu.async_copy(y_hbm.at[sb_b], rows_b, gs_b)

    def body2(k, carry):
        i0 = 1 + 2 * k
        i1 = 2 + 2 * k
        stage_b(i0, rows0, sb0, gs0, ss0, xs0)
        stage_b(i1, rows1, sb1, gs1, ss1, xs1)
        drain_b(i0, rows0, sb0, gs0, ss0, xs0)
        drain_b(i1, rows1, sb1, gs1, ss1, xs1)
        return carry

    lax.fori_loop(0, (NBATCH - 1) // 2, body2, 0)
    plsc.subcore_barrier()
    ostart = pl.multiple_of(c * N + s * ZR, 8)

    @pl.when(s < NS - 1)
    def _():
        pltpu.sync_copy(acc_sp.at[pl.ds(start, ZR)],
                        out_hbm.at[pl.ds(ostart, ZR)])

    @pl.when(s == NS - 1)
    def _():
        last = N - (NS - 1) * ZR
        pltpu.sync_copy(acc_sp.at[pl.ds(start, last)],
                        out_hbm.at[pl.ds(ostart, last)])


def _sc_agg1_body(y_hbm, src_hbm, dst3_hbm, nrm_hbm, zn_hbm, out_hbm,
                  y_v, sbuf, nbuf, dbuf2, vbuf2, stage, acc_sp, ss):
    c = lax.axis_index("c")
    s = lax.axis_index("s")
    wid = s * NC + c
    base = pl.multiple_of(wid * EPW, 8)
    pltpu.sync_copy(y_hbm, y_v)
    pltpu.sync_copy(src_hbm.at[pl.ds(base, EPW)], sbuf)
    pltpu.sync_copy(nrm_hbm.at[pl.ds(base, EPW)], nbuf)
    pltpu.sync_copy(dst3_hbm.at[wid], dbuf2)

    @pl.when(s == 0)
    def _():
        pltpu.sync_copy(zn_hbm, stage)
        pltpu.sync_copy(stage, acc_sp)

    plsc.subcore_barrier()

    def chunk(ch, carry):
        for j in range(5):
            i = ch * 5 + j
            for g in range(KB // 16):
                sl = pl.ds(pl.multiple_of(i * KB + g * 16, 16), 16)
                osl = pl.ds(g * 16, 16)
                vbuf2[i, osl] = plsc.load_gather(y_v, [sbuf[sl]]) * nbuf[sl]
            pltpu.async_copy(vbuf2.at[i], acc_sp.at[dbuf2.at[i]], ss, add=True)
        for j in range(5):
            i = ch * 5 + j
            pltpu.make_async_copy(vbuf2.at[i], acc_sp.at[dbuf2.at[i]],
                                  ss).wait()
        return carry

    lax.fori_loop(0, NBATCH // 5, chunk, 0)
    plsc.subcore_barrier()

    @pl.when(s == 0)
    def _():
        pltpu.sync_copy(acc_sp, stage)
        pltpu.sync_copy(stage, out_hbm.at[pl.ds(pl.multiple_of(c * N, 8), N)])


_sc_deg = functools.partial(
    pl.kernel, _sc_deg_body, mesh=_MESH,
    compiler_params=_SC_PARAMS,
    out_type=jax.ShapeDtypeStruct((NC * N,), jnp.float32),
    scratch_types=[
        pltpu.VMEM((NBATCH, KB), jnp.int32),
        pltpu.VMEM((EPW,), jnp.float32),
        pltpu.VMEM((N,), jnp.float32),
        pltpu.VMEM_SHARED((N,), jnp.float32),
        pltpu.SemaphoreType.DMA,
    ],
)

_sc_norm = functools.partial(
    pl.kernel, _sc_norm_body, mesh=_MESH,
    compiler_params=_SC_PARAMS,
    out_type=jax.ShapeDtypeStruct((E,), jnp.float32),
    scratch_types=[
        pltpu.VMEM((N,), jnp.float32),
        pltpu.VMEM((EPW,), jnp.int32),
        pltpu.VMEM((EPW,), jnp.int32),
        pltpu.VMEM((EPW,), jnp.float32),
        pltpu.VMEM((EPW,), jnp.float32),
    ],
)

_sc_agg = functools.partial(
    pl.kernel, _sc_agg_body, mesh=_MESH,
    compiler_params=_SC_PARAMS,
    out_type=jax.ShapeDtypeStruct((NC * N, 128), jnp.float32),
    scratch_types=[
        pltpu.VMEM((EPW,), jnp.float32),
        pltpu.VMEM((NBATCH, KB), jnp.int32),
        pltpu.VMEM((KB,), jnp.int32),
        pltpu.VMEM((KB,), jnp.int32),
        pltpu.VMEM((KB, 128), jnp.float32),
        pltpu.VMEM((KB, 128), jnp.float32),
        pltpu.VMEM_SHARED((N, 128), jnp.float32),
        pltpu.SemaphoreType.DMA,
        pltpu.SemaphoreType.DMA,
        pltpu.SemaphoreType.DMA,
        pltpu.SemaphoreType.DMA,
        pltpu.SemaphoreType.DMA,
        pltpu.SemaphoreType.DMA,
    ],
)

_sc_agg1 = functools.partial(
    pl.kernel, _sc_agg1_body, mesh=_MESH,
    compiler_params=_SC_PARAMS,
    out_type=jax.ShapeDtypeStruct((NC * N,), jnp.float32),
    scratch_types=[
        pltpu.VMEM((N,), jnp.float32),
        pltpu.VMEM((EPW,), jnp.int32),
        pltpu.VMEM((EPW,), jnp.float32),
        pltpu.VMEM((NBATCH, KB), jnp.int32),
        pltpu.VMEM((NBATCH, KB), jnp.float32),
        pltpu.VMEM((N,), jnp.float32),
        pltpu.VMEM_SHARED((N,), jnp.float32),
        pltpu.SemaphoreType.DMA,
    ],
)


def _ew_body(ef_ref, m_ref, o_ref):
    o_ref[...] = jnp.dot(ef_ref[...], m_ref[...],
                         preferred_element_type=jnp.float32, precision=_HIGH)


def _dinv_body(degp_ref, o_ref):
    deg = degp_ref[0:1, :] + degp_ref[1:2, :] + 1.0
    o_ref[...] = jnp.where(deg > 0, lax.rsqrt(deg), 0.0)


def _l1_body(p0, p1, xr, dv, w1, b1r, w2, o):
    d2 = dv[...] * dv[...]
    z = p0[...] + p1[...] + d2 * xr[...]
    h = jnp.dot(z, w1[...], preferred_element_type=jnp.float32)
    h = jnp.maximum(h + b1r[...], 0.0)
    o[...] = jnp.dot(h, w2[...], preferred_element_type=jnp.float32)


def _l2_body(p0, p1, xr, dv, b2r, w3, o):
    d2 = dv[...] * dv[...]
    h = jnp.maximum(p0[...] + p1[...] + d2 * xr[...] + b2r[...], 0.0)
    o[...] = jnp.dot(h, w3[...], preferred_element_type=jnp.float32)


def _out_body(p3, xw3r, dv, b3r, o):
    o[...] = (p3[0:1, :] + p3[1:2, :]
              + dv[...] * dv[...] * xw3r[...] + b3r[...])


def kernel(x, edge_index, edge_feature, W1, b1, W2, b2, W3, b3):
    src = edge_index[0]
    dst = edge_index[1]
    dst3 = dst.reshape(NW, NBATCH, KB)
    zn = jnp.zeros((N,), jnp.float32)
    zrows = jnp.zeros((N - (NS - 1) * ZR, 128), jnp.float32)

    ef2 = edge_feature.reshape(E // 8, 128)
    m = jnp.repeat(jnp.eye(8, dtype=jnp.float32), 16, axis=0) * (1.0 / 16.0)
    ew8 = pl.pallas_call(
        _ew_body,
        grid=(10,),
        in_specs=[pl.BlockSpec((E // 80, 128), lambda i: (i, 0)),
                  pl.BlockSpec((128, 8), lambda i: (0, 0))],
        out_specs=pl.BlockSpec((E // 80, 8), lambda i: (i, 0)),
        out_shape=jax.ShapeDtypeStruct((E // 8, 8), jnp.float32),
    )(ef2, m)
    ew = ew8.reshape(E)

    degp = _sc_deg()(dst3, ew, zn).reshape(NC, N)
    dinv2d = pl.pallas_call(
        _dinv_body,
        out_shape=jax.ShapeDtypeStruct((1, N), jnp.float32),
    )(degp)
    dinv = dinv2d.reshape(N)
    dvcol = dinv2d.reshape(N, 1)

    nrm = _sc_norm()(src, dst, ew, dinv)

    p1_ = _sc_agg()(x, src, dst3, nrm, zrows)
    b1r = b1.reshape(1, 256)
    b2r = b2.reshape(1, 128)
    xw2 = pl.pallas_call(
        _l1_body,
        grid=(10,),
        in_specs=[pl.BlockSpec((1000, 128), lambda i: (i, 0)),
                  pl.BlockSpec((1000, 128), lambda i: (i, 0)),
                  pl.BlockSpec((1000, 128), lambda i: (i, 0)),
                  pl.BlockSpec((1000, 1), lambda i: (i, 0)),
                  pl.BlockSpec((128, 256), lambda i: (0, 0)),
                  pl.BlockSpec((1, 256), lambda i: (0, 0)),
                  pl.BlockSpec((256, 128), lambda i: (0, 0))],
        out_specs=pl.BlockSpec((1000, 128), lambda i: (i, 0)),
        out_shape=jax.ShapeDtypeStruct((N, 128), jnp.float32),
    )(p1_[:N], p1_[N:], x, dvcol, W1, b1r, W2)

    p2_ = _sc_agg()(xw2, src, dst3, nrm, zrows)
    xw3 = pl.pallas_call(
        _l2_body,
        grid=(10,),
        in_specs=[pl.BlockSpec((1000, 128), lambda i: (i, 0)),
                  pl.BlockSpec((1000, 128), lambda i: (i, 0)),
                  pl.BlockSpec((1000, 128), lambda i: (i, 0)),
                  pl.BlockSpec((1000, 1), lambda i: (i, 0)),
                  pl.BlockSpec((1, 128), lambda i: (0, 0)),
                  pl.BlockSpec((128, 1), lambda i: (0, 0))],
        out_specs=pl.BlockSpec((1000, 1), lambda i: (i, 0)),
        out_shape=jax.ShapeDtypeStruct((N, 1), jnp.float32),
    )(p2_[:N], p2_[N:], xw2, dvcol, b2r, W3)

    p3 = _sc_agg1()(xw3.reshape(N), src, dst3, nrm, zn).reshape(NC, N)
    b3r = b3.reshape(1, 1)
    out2d = pl.pallas_call(
        _out_body,
        out_shape=jax.ShapeDtypeStruct((1, N), jnp.float32),
    )(p3, xw3.reshape(1, N), dinv2d, b3r)
    return out2d.reshape(N)


# speedup vs baseline: 22.3387x; 1.0195x over previous
import functools

import jax
import jax.numpy as jnp
from jax import lax
from jax.experimental import pallas as pl
from jax.experimental.pallas import tpu as pltpu
from jax.experimental.pallas import tpu_sc as plsc

N = 10000
E = 320000
NC, NS = 2, 16
NW = NC * NS
EPW = E // NW
KB = 80
NBATCH = EPW // KB
ZR = 624

_MESH = plsc.VectorSubcoreMesh(core_axis_name="c", subcore_axis_name="s")
_SC_PARAMS = pltpu.CompilerParams(needs_layout_passes=False)
_HIGH = jax.lax.Precision.HIGHEST


def _sc_deg_body(dst3_hbm, ew_hbm, zn_hbm, out_hbm, dbuf2, wbuf, stage,
                 acc_sp, ss):
    c = lax.axis_index("c")
    s = lax.axis_index("s")
    wid = s * NC + c

    @pl.when(s == 0)
    def _():
        pltpu.sync_copy(zn_hbm, stage)
        pltpu.sync_copy(stage, acc_sp)

    pltpu.sync_copy(dst3_hbm.at[wid], dbuf2)
    pltpu.sync_copy(ew_hbm.at[pl.ds(pl.multiple_of(wid * EPW, 8), EPW)], wbuf)
    plsc.subcore_barrier()

    def chunk(ch, carry):
        for j in range(5):
            i = ch * 5 + j
            off = pl.multiple_of(i * KB, 8)
            pltpu.async_copy(wbuf.at[pl.ds(off, KB)], acc_sp.at[dbuf2.at[i]],
                             ss, add=True)
        for j in range(5):
            i = ch * 5 + j
            off = pl.multiple_of(i * KB, 8)
            pltpu.make_async_copy(wbuf.at[pl.ds(off, KB)],
                                  acc_sp.at[dbuf2.at[i]], ss).wait()
        return carry

    lax.fori_loop(0, NBATCH // 5, chunk, 0)
    plsc.subcore_barrier()

    @pl.when(s == 0)
    def _():
        pltpu.sync_copy(acc_sp, stage)
        pltpu.sync_copy(stage, out_hbm.at[pl.ds(pl.multiple_of(c * N, 8), N)])


def _sc_norm_body(src_hbm, dst_hbm, ew_hbm, dinv_hbm, out_hbm,
                  dv, sbuf, dbuf, wbuf, nbuf):
    c = lax.axis_index("c")
    s = lax.axis_index("s")
    wid = s * NC + c
    base = pl.multiple_of(wid * EPW, 8)
    pltpu.sync_copy(dinv_hbm, dv)
    pltpu.sync_copy(src_hbm.at[pl.ds(base, EPW)], sbuf)
    pltpu.sync_copy(dst_hbm.at[pl.ds(base, EPW)], dbuf)
    pltpu.sync_copy(ew_hbm.at[pl.ds(base, EPW)], wbuf)

    @plsc.parallel_loop(0, EPW // 16, unroll=4)
    def _(m):
        sl = pl.ds(pl.multiple_of(m * 16, 16), 16)
        a = plsc.load_gather(dv, [sbuf[sl]])
        b = plsc.load_gather(dv, [dbuf[sl]])
        nbuf[sl] = a * wbuf[sl] * b

    pltpu.sync_copy(nbuf, out_hbm.at[pl.ds(base, EPW)])


def _sc_agg_body(y_hbm, src_hbm, dst3_hbm, nrm_hbm, z_hbm, out_hbm,
                 nbuf, dbuf2, sb0, sb1, rows0, rows1, acc_sp,
                 gs0, gs1, ss0, ss1, xs0, xs1):
    c = lax.axis_index("c")
    s = lax.axis_index("s")
    wid = s * NC + c
    base = pl.multiple_of(wid * EPW, 8)
    pltpu.sync_copy(nrm_hbm.at[pl.ds(base, EPW)], nbuf)
    pltpu.sync_copy(dst3_hbm.at[wid], dbuf2)
    start = pl.multiple_of(s * ZR, 8)

    @pl.when(s < NS - 1)
    def _():
        pltpu.sync_copy(z_hbm.at[pl.ds(0, ZR)], acc_sp.at[pl.ds(start, ZR)])

    @pl.when(s == NS - 1)
    def _():
        pltpu.sync_copy(z_hbm, acc_sp.at[pl.ds(start, N - (NS - 1) * ZR)])

    plsc.subcore_barrier()

    def sidx(i):
        return src_hbm.at[pl.ds(pl.multiple_of(base + i * KB, 8), KB)]

    def scale(rows_b, i):
        @plsc.parallel_loop(0, KB, unroll=4)
        def _(r):
            nb = plsc.load_gather(
                nbuf, [jnp.full((16,), i * KB + r, jnp.int32)])
            for cc in range(8):
                sl = pl.ds(cc * 16, 16)
                rows_b[r, sl] = rows_b[r, sl] * nb

    pltpu.sync_copy(sidx(0), sb0)
    pltpu.async_copy(y_hbm.at[sb0], rows0, gs0).wait()
    scale(rows0, 0)
    pltpu.sync_copy(rows0, acc_sp.at[dbuf2.at[0]], add=True)

    pltpu.sync_copy(sidx(1), sb0)
    pltpu.sync_copy(sidx(2), sb1)
    pltpu.async_copy(y_hbm.at[sb0], rows0, gs0)
    pltpu.async_copy(y_hbm.at[sb1], rows1, gs1)

    def stage_b(i, rows_b, sb_b, gs_b, ss_b, xs_b):
        pltpu.make_async_copy(y_hbm.at[sb_b], rows_b, gs_b).wait()

        @pl.when(i + 2 < NBATCH)
        def _():
            pltpu.async_copy(sidx(i + 2), sb_b, xs_b)

        scale(rows_b, i)
        pltpu.async_copy(rows_b, acc_sp.at[dbuf2.at[i]], ss_b, add=True)

    def drain_b(i, rows_b, sb_b, gs_b, ss_b, xs_b):
        pltpu.make_async_copy(rows_b, acc_sp.at[dbuf2.at[i]], ss_b).wait()

        @pl.when(i + 2 < NBATCH)
        def _():
            pltpu.make_async_copy(sidx(i + 2), sb_b, xs_b).wait()
            pltpu.async_copy(y_hbm.at[sb_b], rows_b, gs_b)

    def body2(k, carry):
        i0 = 1 + 2 * k
        i1 = 2 + 2 * k
        stage_b(i0, rows0, sb0, gs0, ss0, xs0)
        stage_b(i1, rows1, sb1, gs1, ss1, xs1)
        drain_b(i0, rows0, sb0, gs0, ss0, xs0)
        drain_b(i1, rows1, sb1, gs1, ss1, xs1)
        return carry

    lax.fori_loop(0, (NBATCH - 1) // 2, body2, 0)
    plsc.subcore_barrier()
    ostart = pl.multiple_of(c * N + s * ZR, 8)

    @pl.when(s < NS - 1)
    def _():
        pltpu.sync_copy(acc_sp.at[pl.ds(start, ZR)],
                        out_hbm.at[pl.ds(ostart, ZR)])

    @pl.when(s == NS - 1)
    def _():
        last = N - (NS - 1) * ZR
        pltpu.sync_copy(acc_sp.at[pl.ds(start, last)],
                        out_hbm.at[pl.ds(ostart, last)])


def _sc_agg1_body(y_hbm, src_hbm, dst3_hbm, nrm_hbm, zn_hbm, out_hbm,
                  y_v, sbuf, nbuf, dbuf2, vbuf2, stage, acc_sp, ss):
    c = lax.axis_index("c")
    s = lax.axis_index("s")
    wid = s * NC + c
    base = pl.multiple_of(wid * EPW, 8)
    pltpu.sync_copy(y_hbm, y_v)
    pltpu.sync_copy(src_hbm.at[pl.ds(base, EPW)], sbuf)
    pltpu.sync_copy(nrm_hbm.at[pl.ds(base, EPW)], nbuf)
    pltpu.sync_copy(dst3_hbm.at[wid], dbuf2)

    @pl.when(s == 0)
    def _():
        pltpu.sync_copy(zn_hbm, stage)
        pltpu.sync_copy(stage, acc_sp)

    plsc.subcore_barrier()

    def chunk(ch, carry):
        for j in range(5):
            i = ch * 5 + j
            for g in range(KB // 16):
                sl = pl.ds(pl.multiple_of(i * KB + g * 16, 16), 16)
                osl = pl.ds(g * 16, 16)
                vbuf2[i, osl] = plsc.load_gather(y_v, [sbuf[sl]]) * nbuf[sl]
            pltpu.async_copy(vbuf2.at[i], acc_sp.at[dbuf2.at[i]], ss, add=True)
        for j in range(5):
            i = ch * 5 + j
            pltpu.make_async_copy(vbuf2.at[i], acc_sp.at[dbuf2.at[i]],
                                  ss).wait()
        return carry

    lax.fori_loop(0, NBATCH // 5, chunk, 0)
    plsc.subcore_barrier()

    @pl.when(s == 0)
    def _():
        pltpu.sync_copy(acc_sp, stage)
        pltpu.sync_copy(stage, out_hbm.at[pl.ds(pl.multiple_of(c * N, 8), N)])


_sc_deg = functools.partial(
    pl.kernel, _sc_deg_body, mesh=_MESH,
    compiler_params=_SC_PARAMS,
    out_type=jax.ShapeDtypeStruct((NC * N,), jnp.float32),
    scratch_types=[
        pltpu.VMEM((NBATCH, KB), jnp.int32),
        pltpu.VMEM((EPW,), jnp.float32),
        pltpu.VMEM((N,), jnp.float32),
        pltpu.VMEM_SHARED((N,), jnp.float32),
        pltpu.SemaphoreType.DMA,
    ],
)

_sc_norm = functools.partial(
    pl.kernel, _sc_norm_body, mesh=_MESH,
    compiler_params=_SC_PARAMS,
    out_type=jax.ShapeDtypeStruct((E,), jnp.float32),
    scratch_types=[
        pltpu.VMEM((N,), jnp.float32),
        pltpu.VMEM((EPW,), jnp.int32),
        pltpu.VMEM((EPW,), jnp.int32),
        pltpu.VMEM((EPW,), jnp.float32),
        pltpu.VMEM((EPW,), jnp.float32),
    ],
)

_sc_agg = functools.partial(
    pl.kernel, _sc_agg_body, mesh=_MESH,
    compiler_params=_SC_PARAMS,
    out_type=jax.ShapeDtypeStruct((NC * N, 128), jnp.float32),
    scratch_types=[
        pltpu.VMEM((EPW,), jnp.float32),
        pltpu.VMEM((NBATCH, KB), jnp.int32),
        pltpu.VMEM((KB,), jnp.int32),
        pltpu.VMEM((KB,), jnp.int32),
        pltpu.VMEM((KB, 128), jnp.float32),
        pltpu.VMEM((KB, 128), jnp.float32),
        pltpu.VMEM_SHARED((N, 128), jnp.float32),
        pltpu.SemaphoreType.DMA,
        pltpu.SemaphoreType.DMA,
        pltpu.SemaphoreType.DMA,
        pltpu.SemaphoreType.DMA,
        pltpu.SemaphoreType.DMA,
        pltpu.SemaphoreType.DMA,
    ],
)

_sc_agg1 = functools.partial(
    pl.kernel, _sc_agg1_body, mesh=_MESH,
    compiler_params=_SC_PARAMS,
    out_type=jax.ShapeDtypeStruct((NC * N,), jnp.float32),
    scratch_types=[
        pltpu.VMEM((N,), jnp.float32),
        pltpu.VMEM((EPW,), jnp.int32),
        pltpu.VMEM((EPW,), jnp.float32),
        pltpu.VMEM((NBATCH, KB), jnp.int32),
        pltpu.VMEM((NBATCH, KB), jnp.float32),
        pltpu.VMEM((N,), jnp.float32),
        pltpu.VMEM_SHARED((N,), jnp.float32),
        pltpu.SemaphoreType.DMA,
    ],
)


def _ew_body(ef_ref, m_ref, o_ref):
    o_ref[...] = jnp.dot(ef_ref[...], m_ref[...],
                         preferred_element_type=jnp.float32, precision=_HIGH)


def _dinv_body(degp_ref, o_ref):
    deg = degp_ref[0:1, :] + degp_ref[1:2, :] + 1.0
    o_ref[...] = jnp.where(deg > 0, lax.rsqrt(deg), 0.0)


def _l1_body(p0, p1, xr, dv, w1, b1r, w2, o):
    d2 = dv[...] * dv[...]
    z = p0[...] + p1[...] + d2 * xr[...]
    h = jnp.dot(z, w1[...], preferred_element_type=jnp.float32)
    h = jnp.maximum(h + b1r[...], 0.0)
    o[...] = jnp.dot(h, w2[...], preferred_element_type=jnp.float32)


def _l2_body(p0, p1, xr, dv, b2r, w3, o):
    d2 = dv[...] * dv[...]
    h = jnp.maximum(p0[...] + p1[...] + d2 * xr[...] + b2r[...], 0.0)
    o[...] = jnp.dot(h, w3[...], preferred_element_type=jnp.float32)


def _out_body(p3, xw3r, dv, b3r, o):
    o[...] = (p3[0:1, :] + p3[1:2, :]
              + dv[...] * dv[...] * xw3r[...] + b3r[...])


def kernel(x, edge_index, edge_feature, W1, b1, W2, b2, W3, b3):
    src = edge_index[0]
    dst = edge_index[1]
    dst3 = dst.reshape(NW, NBATCH, KB)
    zn = jnp.zeros((N,), jnp.float32)
    zrows = jnp.zeros((N - (NS - 1) * ZR, 128), jnp.float32)

    ef2 = edge_feature.reshape(E // 8, 128)
    m = jnp.repeat(jnp.eye(8, dtype=jnp.float32), 16, axis=0) * (1.0 / 16.0)
    ew8 = pl.pallas_call(
        _ew_body,
        grid=(10,),
        in_specs=[pl.BlockSpec((E // 80, 128), lambda i: (i, 0)),
                  pl.BlockSpec((128, 8), lambda i: (0, 0))],
        out_specs=pl.BlockSpec((E // 80, 8), lambda i: (i, 0)),
        out_shape=jax.ShapeDtypeStruct((E // 8, 8), jnp.float32),
    )(ef2, m)
    ew = ew8.reshape(E)

    degp = _sc_deg()(dst3, ew, zn).reshape(NC, N)
    dinv2d = pl.pallas_call(
        _dinv_body,
        out_shape=jax.ShapeDtypeStruct((1, N), jnp.float32),
    )(degp)
    dinv = dinv2d.reshape(N)
    dvcol = dinv2d.reshape(N, 1)

    nrm = _sc_norm()(src, dst, ew, dinv)

    p1_ = _sc_agg()(x, src, dst3, nrm, zrows)
    b1r = b1.reshape(1, 256)
    b2r = b2.reshape(1, 128)
    xw2 = pl.pallas_call(
        _l1_body,
        grid=(10,),
        in_specs=[pl.BlockSpec((1000, 128), lambda i: (i, 0)),
                  pl.BlockSpec((1000, 128), lambda i: (i + 10, 0)),
                  pl.BlockSpec((1000, 128), lambda i: (i, 0)),
                  pl.BlockSpec((1000, 1), lambda i: (i, 0)),
                  pl.BlockSpec((128, 256), lambda i: (0, 0)),
                  pl.BlockSpec((1, 256), lambda i: (0, 0)),
                  pl.BlockSpec((256, 128), lambda i: (0, 0))],
        out_specs=pl.BlockSpec((1000, 128), lambda i: (i, 0)),
        out_shape=jax.ShapeDtypeStruct((N, 128), jnp.float32),
    )(p1_, p1_, x, dvcol, W1, b1r, W2)

    p2_ = _sc_agg()(xw2, src, dst3, nrm, zrows)
    xw3 = pl.pallas_call(
        _l2_body,
        grid=(10,),
        in_specs=[pl.BlockSpec((1000, 128), lambda i: (i, 0)),
                  pl.BlockSpec((1000, 128), lambda i: (i + 10, 0)),
                  pl.BlockSpec((1000, 128), lambda i: (i, 0)),
                  pl.BlockSpec((1000, 1), lambda i: (i, 0)),
                  pl.BlockSpec((1, 128), lambda i: (0, 0)),
                  pl.BlockSpec((128, 1), lambda i: (0, 0))],
        out_specs=pl.BlockSpec((1000, 1), lambda i: (i, 0)),
        out_shape=jax.ShapeDtypeStruct((N, 1), jnp.float32),
    )(p2_, p2_, xw2, dvcol, b2r, W3)

    p3 = _sc_agg1()(xw3.reshape(N), src, dst3, nrm, zn).reshape(NC, N)
    b3r = b3.reshape(1, 1)
    out2d = pl.pallas_call(
        _out_body,
        out_shape=jax.ShapeDtypeStruct((1, N), jnp.float32),
    )(p3, xw3.reshape(1, N), dinv2d, b3r)
    return out2d.reshape(N)


# speedup vs baseline: 24.1324x; 1.0803x over previous
import functools

import jax
import jax.numpy as jnp
from jax import lax
from jax.experimental import pallas as pl
from jax.experimental.pallas import tpu as pltpu
from jax.experimental.pallas import tpu_sc as plsc

N = 10000
E = 320000
NC, NS = 2, 16
NW = NC * NS
EPW = E // NW
KB = 80
NBATCH = EPW // KB
ZR = 624

_MESH = plsc.VectorSubcoreMesh(core_axis_name="c", subcore_axis_name="s")
_SC_PARAMS = pltpu.CompilerParams(needs_layout_passes=False)
_HIGH = jax.lax.Precision.HIGHEST


def _sc_deg_body(dst3_hbm, ew_hbm, zn_hbm, out_hbm, dbuf2, wbuf, stage,
                 acc_sp, ss):
    c = lax.axis_index("c")
    s = lax.axis_index("s")
    wid = s * NC + c

    @pl.when(s == 0)
    def _():
        pltpu.sync_copy(zn_hbm, stage)
        pltpu.sync_copy(stage, acc_sp)

    pltpu.sync_copy(dst3_hbm.at[wid], dbuf2)
    pltpu.sync_copy(ew_hbm.at[pl.ds(pl.multiple_of(wid * EPW, 8), EPW)], wbuf)
    plsc.subcore_barrier()

    def chunk(ch, carry):
        for j in range(5):
            i = ch * 5 + j
            off = pl.multiple_of(i * KB, 8)
            pltpu.async_copy(wbuf.at[pl.ds(off, KB)], acc_sp.at[dbuf2.at[i]],
                             ss, add=True)
        for j in range(5):
            i = ch * 5 + j
            off = pl.multiple_of(i * KB, 8)
            pltpu.make_async_copy(wbuf.at[pl.ds(off, KB)],
                                  acc_sp.at[dbuf2.at[i]], ss).wait()
        return carry

    lax.fori_loop(0, NBATCH // 5, chunk, 0)
    plsc.subcore_barrier()

    @pl.when(s == 0)
    def _():
        pltpu.sync_copy(acc_sp, stage)
        pltpu.sync_copy(stage, out_hbm.at[pl.ds(pl.multiple_of(c * N, 8), N)])


def _sc_norm_body(src_hbm, dst_hbm, ew_hbm, dinv_hbm, out_hbm,
                  dv, sbuf, dbuf, wbuf, nbuf):
    c = lax.axis_index("c")
    s = lax.axis_index("s")
    wid = s * NC + c
    base = pl.multiple_of(wid * EPW, 8)
    pltpu.sync_copy(dinv_hbm, dv)
    pltpu.sync_copy(src_hbm.at[pl.ds(base, EPW)], sbuf)
    pltpu.sync_copy(dst_hbm.at[pl.ds(base, EPW)], dbuf)
    pltpu.sync_copy(ew_hbm.at[pl.ds(base, EPW)], wbuf)

    @plsc.parallel_loop(0, EPW // 16, unroll=4)
    def _(m):
        sl = pl.ds(pl.multiple_of(m * 16, 16), 16)
        a = plsc.load_gather(dv, [sbuf[sl]])
        b = plsc.load_gather(dv, [dbuf[sl]])
        nbuf[sl] = a * wbuf[sl] * b

    pltpu.sync_copy(nbuf, out_hbm.at[pl.ds(base, EPW)])


def _sc_agg_body(y_hbm, src_hbm, dst_hbm, nrm_hbm, z_hbm, out_hbm,
                 db, sb, nb, rows, acc_sp, gs, ss, xs, ns, ds_):
    c = lax.axis_index("c")
    s = lax.axis_index("s")
    wid = s * NC + c
    base = pl.multiple_of(wid * EPW, 8)
    start = pl.multiple_of(s * ZR, 8)

    @pl.when(s < NS - 1)
    def _():
        pltpu.sync_copy(z_hbm.at[pl.ds(0, ZR)], acc_sp.at[pl.ds(start, ZR)])

    @pl.when(s == NS - 1)
    def _():
        pltpu.sync_copy(z_hbm, acc_sp.at[pl.ds(start, N - (NS - 1) * ZR)])

    plsc.subcore_barrier()

    def sidx(i):
        return src_hbm.at[pl.ds(pl.multiple_of(base + i * KB, 8), KB)]

    def nidx(i):
        return nrm_hbm.at[pl.ds(pl.multiple_of(base + i * KB, 8), KB)]

    def didx(i):
        return dst_hbm.at[pl.ds(pl.multiple_of(base + i * KB, 8), KB)]

    def scale(rows_b, nb_b):
        @plsc.parallel_loop(0, KB, unroll=2)
        def _(r):
            nv = plsc.load_gather(nb_b, [jnp.full((16,), r, jnp.int32)])
            for cc in range(8):
                sl = pl.ds(cc * 16, 16)
                rows_b[r, sl] = rows_b[r, sl] * nv

    pltpu.sync_copy(sidx(0), sb[0])
    pltpu.sync_copy(nidx(0), nb[0])
    pltpu.sync_copy(didx(0), db[0])
    pltpu.async_copy(y_hbm.at[sb[0]], rows[0], gs[0]).wait()
    scale(rows[0], nb[0])
    pltpu.sync_copy(rows[0], acc_sp.at[db[0]], add=True)

    for b in range(4):
        pltpu.sync_copy(sidx(1 + b), sb[b])
        pltpu.sync_copy(nidx(1 + b), nb[b])
        pltpu.sync_copy(didx(1 + b), db[b])
        pltpu.async_copy(y_hbm.at[sb[b]], rows[b], gs[b])

    def stage_b(i, b):
        pltpu.make_async_copy(y_hbm.at[sb[b]], rows[b], gs[b]).wait()

        @pl.when(i + 4 < NBATCH)
        def _():
            pltpu.async_copy(sidx(i + 4), sb[b], xs[b])

        scale(rows[b], nb[b])

        @pl.when(i + 4 < NBATCH)
        def _():
            pltpu.async_copy(nidx(i + 4), nb[b], ns[b])

        pltpu.async_copy(rows[b], acc_sp.at[db[b]], ss[b], add=True)

    def drain_b(i, b):
        pltpu.make_async_copy(rows[b], acc_sp.at[db[b]], ss[b]).wait()

        @pl.when(i + 4 < NBATCH)
        def _():
            pltpu.make_async_copy(sidx(i + 4), sb[b], xs[b]).wait()
            pltpu.make_async_copy(nidx(i + 4), nb[b], ns[b]).wait()
            pltpu.async_copy(y_hbm.at[sb[b]], rows[b], gs[b])
            pltpu.async_copy(didx(i + 4), db[b], ds_[b])
            pltpu.make_async_copy(didx(i + 4), db[b], ds_[b]).wait()

    def body4(k, carry):
        i0 = 1 + 4 * k
        for b in range(4):
            stage_b(i0 + b, b)
        for b in range(4):
            drain_b(i0 + b, b)
        return carry

    lax.fori_loop(0, (NBATCH - 1) // 4, body4, 0)
    plsc.subcore_barrier()
    ostart = pl.multiple_of(c * N + s * ZR, 8)

    @pl.when(s < NS - 1)
    def _():
        pltpu.sync_copy(acc_sp.at[pl.ds(start, ZR)],
                        out_hbm.at[pl.ds(ostart, ZR)])

    @pl.when(s == NS - 1)
    def _():
        last = N - (NS - 1) * ZR
        pltpu.sync_copy(acc_sp.at[pl.ds(start, last)],
                        out_hbm.at[pl.ds(ostart, last)])


def _sc_agg1_body(y_hbm, src_hbm, dst3_hbm, nrm_hbm, zn_hbm, out_hbm,
                  y_v, sbuf, nbuf, dbuf2, vbuf2, stage, acc_sp, ss):
    c = lax.axis_index("c")
    s = lax.axis_index("s")
    wid = s * NC + c
    base = pl.multiple_of(wid * EPW, 8)
    pltpu.sync_copy(y_hbm, y_v)
    pltpu.sync_copy(src_hbm.at[pl.ds(base, EPW)], sbuf)
    pltpu.sync_copy(nrm_hbm.at[pl.ds(base, EPW)], nbuf)
    pltpu.sync_copy(dst3_hbm.at[wid], dbuf2)

    @pl.when(s == 0)
    def _():
        pltpu.sync_copy(zn_hbm, stage)
        pltpu.sync_copy(stage, acc_sp)

    plsc.subcore_barrier()

    def chunk(ch, carry):
        for j in range(5):
            i = ch * 5 + j
            for g in range(KB // 16):
                sl = pl.ds(pl.multiple_of(i * KB + g * 16, 16), 16)
                osl = pl.ds(g * 16, 16)
                vbuf2[i, osl] = plsc.load_gather(y_v, [sbuf[sl]]) * nbuf[sl]
            pltpu.async_copy(vbuf2.at[i], acc_sp.at[dbuf2.at[i]], ss, add=True)
        for j in range(5):
            i = ch * 5 + j
            pltpu.make_async_copy(vbuf2.at[i], acc_sp.at[dbuf2.at[i]],
                                  ss).wait()
        return carry

    lax.fori_loop(0, NBATCH // 5, chunk, 0)
    plsc.subcore_barrier()

    @pl.when(s == 0)
    def _():
        pltpu.sync_copy(acc_sp, stage)
        pltpu.sync_copy(stage, out_hbm.at[pl.ds(pl.multiple_of(c * N, 8), N)])


_sc_deg = functools.partial(
    pl.kernel, _sc_deg_body, mesh=_MESH,
    compiler_params=_SC_PARAMS,
    out_type=jax.ShapeDtypeStruct((NC * N,), jnp.float32),
    scratch_types=[
        pltpu.VMEM((NBATCH, KB), jnp.int32),
        pltpu.VMEM((EPW,), jnp.float32),
        pltpu.VMEM((N,), jnp.float32),
        pltpu.VMEM_SHARED((N,), jnp.float32),
        pltpu.SemaphoreType.DMA,
    ],
)

_sc_norm = functools.partial(
    pl.kernel, _sc_norm_body, mesh=_MESH,
    compiler_params=_SC_PARAMS,
    out_type=jax.ShapeDtypeStruct((E,), jnp.float32),
    scratch_types=[
        pltpu.VMEM((N,), jnp.float32),
        pltpu.VMEM((EPW,), jnp.int32),
        pltpu.VMEM((EPW,), jnp.int32),
        pltpu.VMEM((EPW,), jnp.float32),
        pltpu.VMEM((EPW,), jnp.float32),
    ],
)

_sc_agg = functools.partial(
    pl.kernel, _sc_agg_body, mesh=_MESH,
    compiler_params=_SC_PARAMS,
    out_type=jax.ShapeDtypeStruct((NC * N, 128), jnp.float32),
    scratch_types=[
        [pltpu.VMEM((KB,), jnp.int32) for _ in range(4)],
        [pltpu.VMEM((KB,), jnp.int32) for _ in range(4)],
        [pltpu.VMEM((KB,), jnp.float32) for _ in range(4)],
        [pltpu.VMEM((KB, 128), jnp.float32) for _ in range(4)],
        pltpu.VMEM_SHARED((N, 128), jnp.float32),
        [pltpu.SemaphoreType.DMA for _ in range(4)],
        [pltpu.SemaphoreType.DMA for _ in range(4)],
        [pltpu.SemaphoreType.DMA for _ in range(4)],
        [pltpu.SemaphoreType.DMA for _ in range(4)],
        [pltpu.SemaphoreType.DMA for _ in range(4)],
    ],
)

_sc_agg1 = functools.partial(
    pl.kernel, _sc_agg1_body, mesh=_MESH,
    compiler_params=_SC_PARAMS,
    out_type=jax.ShapeDtypeStruct((NC * N,), jnp.float32),
    scratch_types=[
        pltpu.VMEM((N,), jnp.float32),
        pltpu.VMEM((EPW,), jnp.int32),
        pltpu.VMEM((EPW,), jnp.float32),
        pltpu.VMEM((NBATCH, KB), jnp.int32),
        pltpu.VMEM((NBATCH, KB), jnp.float32),
        pltpu.VMEM((N,), jnp.float32),
        pltpu.VMEM_SHARED((N,), jnp.float32),
        pltpu.SemaphoreType.DMA,
    ],
)


def _ew_body(ef_ref, m_ref, o_ref):
    o_ref[...] = jnp.dot(ef_ref[...], m_ref[...],
                         preferred_element_type=jnp.float32, precision=_HIGH)


def _dinv_body(degp_ref, o_ref):
    deg = degp_ref[0:1, :] + degp_ref[1:2, :] + 1.0
    o_ref[...] = jnp.where(deg > 0, lax.rsqrt(deg), 0.0)


def _l1_body(p0, p1, xr, dv, w1, b1r, w2, o):
    d2 = dv[...] * dv[...]
    z = p0[...] + p1[...] + d2 * xr[...]
    h = jnp.dot(z, w1[...], preferred_element_type=jnp.float32)
    h = jnp.maximum(h + b1r[...], 0.0)
    o[...] = jnp.dot(h, w2[...], preferred_element_type=jnp.float32)


def _l2_body(p0, p1, xr, dv, b2r, w3, o):
    d2 = dv[...] * dv[...]
    h = jnp.maximum(p0[...] + p1[...] + d2 * xr[...] + b2r[...], 0.0)
    o[...] = jnp.dot(h, w3[...], preferred_element_type=jnp.float32)


def _out_body(p3, xw3r, dv, b3r, o):
    o[...] = (p3[0:1, :] + p3[1:2, :]
              + dv[...] * dv[...] * xw3r[...] + b3r[...])


def kernel(x, edge_index, edge_feature, W1, b1, W2, b2, W3, b3):
    src = edge_index[0]
    dst = edge_index[1]
    dst3 = dst.reshape(NW, NBATCH, KB)
    zn = jnp.zeros((N,), jnp.float32)
    zrows = jnp.zeros((N - (NS - 1) * ZR, 128), jnp.float32)

    ef2 = edge_feature.reshape(E // 8, 128)
    m = jnp.repeat(jnp.eye(8, dtype=jnp.float32), 16, axis=0) * (1.0 / 16.0)
    ew8 = pl.pallas_call(
        _ew_body,
        grid=(10,),
        in_specs=[pl.BlockSpec((E // 80, 128), lambda i: (i, 0)),
                  pl.BlockSpec((128, 8), lambda i: (0, 0))],
        out_specs=pl.BlockSpec((E // 80, 8), lambda i: (i, 0)),
        out_shape=jax.ShapeDtypeStruct((E // 8, 8), jnp.float32),
    )(ef2, m)
    ew = ew8.reshape(E)

    degp = _sc_deg()(dst3, ew, zn).reshape(NC, N)
    dinv2d = pl.pallas_call(
        _dinv_body,
        out_shape=jax.ShapeDtypeStruct((1, N), jnp.float32),
    )(degp)
    dinv = dinv2d.reshape(N)
    dvcol = dinv2d.reshape(N, 1)

    nrm = _sc_norm()(src, dst, ew, dinv)

    p1_ = _sc_agg()(x, src, dst, nrm, zrows)
    b1r = b1.reshape(1, 256)
    b2r = b2.reshape(1, 128)
    xw2 = pl.pallas_call(
        _l1_body,
        grid=(10,),
        in_specs=[pl.BlockSpec((1000, 128), lambda i: (i, 0)),
                  pl.BlockSpec((1000, 128), lambda i: (i + 10, 0)),
                  pl.BlockSpec((1000, 128), lambda i: (i, 0)),
                  pl.BlockSpec((1000, 1), lambda i: (i, 0)),
                  pl.BlockSpec((128, 256), lambda i: (0, 0)),
                  pl.BlockSpec((1, 256), lambda i: (0, 0)),
                  pl.BlockSpec((256, 128), lambda i: (0, 0))],
        out_specs=pl.BlockSpec((1000, 128), lambda i: (i, 0)),
        out_shape=jax.ShapeDtypeStruct((N, 128), jnp.float32),
    )(p1_, p1_, x, dvcol, W1, b1r, W2)

    p2_ = _sc_agg()(xw2, src, dst, nrm, zrows)
    xw3 = pl.pallas_call(
        _l2_body,
        grid=(10,),
        in_specs=[pl.BlockSpec((1000, 128), lambda i: (i, 0)),
                  pl.BlockSpec((1000, 128), lambda i: (i + 10, 0)),
                  pl.BlockSpec((1000, 128), lambda i: (i, 0)),
                  pl.BlockSpec((1000, 1), lambda i: (i, 0)),
                  pl.BlockSpec((1, 128), lambda i: (0, 0)),
                  pl.BlockSpec((128, 1), lambda i: (0, 0))],
        out_specs=pl.BlockSpec((1000, 1), lambda i: (i, 0)),
        out_shape=jax.ShapeDtypeStruct((N, 1), jnp.float32),
    )(p2_, p2_, xw2, dvcol, b2r, W3)

    p3 = _sc_agg1()(xw3.reshape(N), src, dst3, nrm, zn).reshape(NC, N)
    b3r = b3.reshape(1, 1)
    out2d = pl.pallas_call(
        _out_body,
        out_shape=jax.ShapeDtypeStruct((1, N), jnp.float32),
    )(p3, xw3.reshape(1, N), dinv2d, b3r)
    return out2d.reshape(N)
